# baseline (speedup 1.0000x reference)
import math

import jax
import jax.numpy as jnp
from jax.experimental import pallas as pl
from jax.experimental.pallas import tpu as pltpu

D = 16
L = 768
T = 96
BR = 128
BC = 128
NB = L // BR
NC = L // BC

_K = [math.exp(-0.5), math.exp(-1.0), math.exp(-2.0), math.exp(-4.0)]
_E1 = _K[0] + _K[1] + _K[2] + _K[3]
_E2 = sum(_K[i] * _K[j] for i in range(4) for j in range(i + 1, 4))
_E3 = sum(_K[i] * _K[j] * _K[l]
          for i in range(4) for j in range(i + 1, 4) for l in range(j + 1, 4))
_E4 = _K[0] * _K[1] * _K[2] * _K[3]
_P0, _P1, _P2, _P3 = 4.0, 3.0 * _E1, 2.0 * _E2, _E3


def _lam(s):
    return (s * s + 256.0) / ((s * 16.0) * (s * 16.0))


_TILES = [(rb, cb) for rb in range(NB) for cb in range(NC)
          if (cb + 1) * BC > rb * BR + 1]
NT = len(_TILES)


def _body(rbt_ref, cbt_ref, xl3_ref, xln_ref, xgt3_ref, xgr_ref,
          tokc_ref, tokr_ref, cmc_ref, cmr_ref, dna_ref, rna_ref, ucol_ref,
          ligcol_ref, t2_ref, out_ref, acc_s, acc_pm):
    i = pl.program_id(0)
    f32 = jnp.float32

    @pl.when(i == 0)
    def _init():
        acc_s[...] = jnp.zeros_like(acc_s)
        acc_pm[...] = jnp.zeros_like(acc_pm)

    if True:
        r0 = rbt_ref[i] * BR
        c0 = cbt_ref[i] * BC
        rows = jax.lax.broadcasted_iota(jnp.int32, (BR, 1), 0) + r0
        cols = jax.lax.broadcasted_iota(jnp.int32, (1, BC), 1) + c0

        gr = [jnp.nan_to_num(xgr_ref[pl.ds(r0, BR), c:c + 1])
              for c in range(3)]
        gc = [jnp.nan_to_num(xgt3_ref[c, 0:1, pl.ds(c0, BC)])
              for c in range(3)]
        gd = jnp.sqrt((gr[0] - gc[0]) ** 2 + (gr[1] - gc[1]) ** 2
                      + (gr[2] - gc[2]) ** 2)

        tokr_blk = tokc_ref[pl.ds(r0, BR), :]
        iota_t_lane = jax.lax.broadcasted_iota(jnp.int32, (1, T), 1)
        oh_rows = (tokr_blk == iota_t_lane).astype(f32)
        na_tok = jnp.minimum(dna_ref[...] + rna_ref[...], 1.0)
        na_rows = jnp.sum(oh_rows * na_tok, axis=1, keepdims=True)
        cutoff = jnp.where(na_rows > 0.0, 30.0, 15.0)

        cm_rows = cmc_ref[pl.ds(r0, BR), :]
        pm = jnp.logical_and(gd > 0.0, gd < cutoff).astype(f32)
        pm = pm * cm_rows * cmr_ref[:, pl.ds(c0, BC)]
        pm = pm * (tokr_blk != tokr_ref[:, pl.ds(c0, BC)]).astype(f32)
        pm = pm * (cols > rows).astype(f32)
        acc_pm[:, pl.ds(c0, BC)] = (acc_pm[:, pl.ds(c0, BC)]
                                    + jnp.sum(pm, axis=0, keepdims=True))

        for d in range(D):
            pr = [xln_ref[d, pl.ds(r0, BR), c:c + 1]
                  for c in range(3)]
            co = [xl3_ref[c, d:d + 1, pl.ds(c0, BC)] for c in range(3)]
            dist2 = ((pr[0] - co[0]) ** 2 + (pr[1] - co[1]) ** 2
                     + (pr[2] - co[2]) ** 2)
            pred = jnp.sqrt(dist2)
            delta = jnp.abs(pred - gd + 1e-6)
            e = jnp.exp(delta)
            s = (1.0 / (1.0 + e * _K[0]) + 1.0 / (1.0 + e * _K[1])
                 + 1.0 / (1.0 + e * _K[2]) + 1.0 / (1.0 + e * _K[3]))
            acc_s[d:d + 1, pl.ds(c0, BC)] = (
                acc_s[d:d + 1, pl.ds(c0, BC)]
                + jnp.sum(s * pm, axis=0, keepdims=True))

    @pl.when(i == NT - 1)
    def _final():
        s_tot = jnp.sum(acc_s[...], axis=1, keepdims=True)
        pm_tot = jnp.sum(acc_pm[...])
        lddt = 0.25 * s_tot / (pm_tot + 1e-6)
        lddt_mean = jnp.mean(1.0 - lddt)

        iota_t_sub = jax.lax.broadcasted_iota(jnp.int32, (T, 1), 0)
        oh_cols = (iota_t_sub == tokr_ref[...]).astype(f32)
        u_atom = jnp.sum(oh_cols * ucol_ref[...], axis=0, keepdims=True)
        lig_atom = jnp.sum(oh_cols * ligcol_ref[...], axis=0, keepdims=True)

        cm = cmr_ref[...]
        t2 = t2_ref[...]
        w = (1.0 + u_atom) * (1.0 + lig_atom) * cm

        diff = [xl3_ref[c] - jnp.nan_to_num(xgt3_ref[c]) for c in range(3)]
        sq = diff[0] ** 2 + diff[1] ** 2 + diff[2] ** 2
        cm_sum = jnp.sum(cm)
        l_mse = (w * sq) / (3.0 * cm_sum + 1e-4)

        any_u = jnp.sum(u_atom) > 0.0
        te = t2 * (1.0 - 0.5 * u_atom)
        lgu = jnp.sum(_lam(te) * l_mse, axis=1, keepdims=True)
        rr = _lam(0.5 * t2) / _lam(t2)
        cmu = jnp.sum(cm * u_atom)
        cmn = jnp.sum(cm * (1.0 - u_atom))
        lgu = lgu * (cm_sum / (rr * cmu + cmn))
        lgp = _lam(t2) * jnp.sum(l_mse, axis=1, keepdims=True)
        lg = jnp.where(any_u, lgu, lgp)

        lp = w * (jnp.abs(diff[0]) + jnp.abs(diff[1]) + jnp.abs(diff[2]))
        usum = jnp.sum(u_atom)
        lpu = (jnp.sum(lp * u_atom, axis=1, keepdims=True)
               / (6.0 * usum + 1e-4)) * _lam(0.5 * t2)
        ltot = jnp.where(any_u, lg + 0.1 * lpu, lg)
        l_mse_total = jnp.mean(jnp.minimum(ltot, 2.0))

        out_ref[...] = jnp.broadcast_to(l_mse_total + lddt_mean, (1, 1))


def kernel(X_L, X_gt_L_in_input_frame, crd_mask_L, t, atom_to_token_map,
           is_original_unindexed_token, is_polar, is_ligand, is_virtual,
           is_sidechain, is_dna, is_rna):
    f32 = jnp.float32
    X_L = X_L.astype(f32)
    X_gt = X_gt_L_in_input_frame.astype(f32)
    xl3 = jnp.transpose(X_L, (2, 0, 1))
    xgt3 = jnp.transpose(X_gt, (2, 0, 1))
    xg0 = X_gt[0]
    tok = atom_to_token_map.astype(jnp.int32)
    tokc = tok.reshape(L, 1)
    tokr = tok.reshape(1, L)
    cm = crd_mask_L.astype(f32)
    cmc = cm.reshape(L, 1)
    cmr = cm.reshape(1, L)
    dna = is_dna.astype(f32).reshape(1, T)
    rna = is_rna.astype(f32).reshape(1, T)
    ucol = is_original_unindexed_token.astype(f32).reshape(T, 1)
    ligcol = is_ligand.astype(f32).reshape(T, 1)
    t2 = t.astype(f32).reshape(D, 1)

    rb_tab = jnp.array([rc[0] for rc in _TILES], dtype=jnp.int32)
    cb_tab = jnp.array([rc[1] for rc in _TILES], dtype=jnp.int32)

    full = lambda s: pl.BlockSpec(s, lambda i, rbt, cbt: (0,) * len(s))
    grid_spec = pltpu.PrefetchScalarGridSpec(
        num_scalar_prefetch=2,
        grid=(NT,),
        in_specs=[
            full((3, D, L)), full((D, L, 3)), full((3, D, L)),
            full((L, 3)),
            full((L, 1)), full((1, L)), full((L, 1)), full((1, L)),
            full((1, T)), full((1, T)), full((T, 1)), full((T, 1)),
            full((D, 1)),
        ],
        out_specs=full((1, 1)),
        scratch_shapes=[
            pltpu.VMEM((D, L), f32),
            pltpu.VMEM((1, L), f32),
        ],
    )
    out = pl.pallas_call(
        _body,
        grid_spec=grid_spec,
        out_shape=jax.ShapeDtypeStruct((1, 1), f32),
    )(rb_tab, cb_tab, xl3, X_L, xgt3, xg0, tokc, tokr, cmc, cmr, dna,
      rna, ucol, ligcol, t2)
    return out[0, 0]


# speedup vs baseline: 1.2100x; 1.2100x over previous
import math

import jax
import jax.numpy as jnp
from jax.experimental import pallas as pl
from jax.experimental.pallas import tpu as pltpu

D = 16
L = 768
T = 96
BR = 128
BC = 128
NB = L // BR
NC = L // BC

_K = [math.exp(-0.5), math.exp(-1.0), math.exp(-2.0), math.exp(-4.0)]
_E1 = _K[0] + _K[1] + _K[2] + _K[3]
_E2 = sum(_K[i] * _K[j] for i in range(4) for j in range(i + 1, 4))
_E3 = sum(_K[i] * _K[j] * _K[l]
          for i in range(4) for j in range(i + 1, 4) for l in range(j + 1, 4))
_E4 = _K[0] * _K[1] * _K[2] * _K[3]
_P0, _P1, _P2, _P3 = 4.0, 3.0 * _E1, 2.0 * _E2, _E3


def _lam(s):
    return (s * s + 256.0) / ((s * 16.0) * (s * 16.0))


_TILES = [(rb, cb) for rb in range(NB) for cb in range(NC)
          if (cb + 1) * BC > rb * BR + 1]
NT = len(_TILES)


def _body(rbt_ref, cbt_ref, xl3_ref, xlr_ref, xgt3_ref, xgr_ref,
          tokc_ref, tokr_ref, cmc_ref, cmr_ref, dna_ref, rna_ref, ucol_ref,
          ligcol_ref, t2_ref, out_ref, acc_s, acc_pm):
    i = pl.program_id(0)
    f32 = jnp.float32

    @pl.when(i == 0)
    def _init():
        acc_s[...] = jnp.zeros_like(acc_s)
        acc_pm[...] = jnp.zeros_like(acc_pm)

    if True:
        r0 = rbt_ref[i] * BR
        c0 = cbt_ref[i] * BC
        rows = jax.lax.broadcasted_iota(jnp.int32, (BR, 1), 0) + r0
        cols = jax.lax.broadcasted_iota(jnp.int32, (1, BC), 1) + c0

        gr = [jnp.nan_to_num(xgr_ref[pl.ds(r0, BR), c:c + 1])
              for c in range(3)]
        gc = [jnp.nan_to_num(xgt3_ref[c, 0:1, pl.ds(c0, BC)])
              for c in range(3)]
        gd = jnp.sqrt((gr[0] - gc[0]) ** 2 + (gr[1] - gc[1]) ** 2
                      + (gr[2] - gc[2]) ** 2)

        tokr_blk = tokc_ref[pl.ds(r0, BR), :]
        iota_t_lane = jax.lax.broadcasted_iota(jnp.int32, (1, T), 1)
        oh_rows = (tokr_blk == iota_t_lane).astype(f32)
        na_tok = jnp.minimum(dna_ref[...] + rna_ref[...], 1.0)
        na_rows = jnp.sum(oh_rows * na_tok, axis=1, keepdims=True)
        cutoff = jnp.where(na_rows > 0.0, 30.0, 15.0)

        cm_rows = cmc_ref[pl.ds(r0, BR), :]
        pm = jnp.logical_and(gd > 0.0, gd < cutoff).astype(f32)
        pm = pm * cm_rows * cmr_ref[:, pl.ds(c0, BC)]
        pm = pm * (tokr_blk != tokr_ref[:, pl.ds(c0, BC)]).astype(f32)
        pm = pm * (cols > rows).astype(f32)
        acc_pm[:, pl.ds(c0, BC)] = (acc_pm[:, pl.ds(c0, BC)]
                                    + jnp.sum(pm, axis=0, keepdims=True))

        for d in range(D):
            pr = [xlr_ref[pl.ds(r0, BR), 3 * d + c:3 * d + c + 1]
                  for c in range(3)]
            co = [xl3_ref[c, d:d + 1, pl.ds(c0, BC)] for c in range(3)]
            dist2 = ((pr[0] - co[0]) ** 2 + (pr[1] - co[1]) ** 2
                     + (pr[2] - co[2]) ** 2)
            pred = jnp.sqrt(dist2)
            delta = jnp.abs(pred - gd + 1e-6)
            e = jnp.exp(delta)
            s = (1.0 / (1.0 + e * _K[0]) + 1.0 / (1.0 + e * _K[1])
                 + 1.0 / (1.0 + e * _K[2]) + 1.0 / (1.0 + e * _K[3]))
            acc_s[d:d + 1, pl.ds(c0, BC)] = (
                acc_s[d:d + 1, pl.ds(c0, BC)]
                + jnp.sum(s * pm, axis=0, keepdims=True))

    @pl.when(i == NT - 1)
    def _final():
        s_tot = jnp.sum(acc_s[...], axis=1, keepdims=True)
        pm_tot = jnp.sum(acc_pm[...])
        lddt = 0.25 * s_tot / (pm_tot + 1e-6)
        lddt_mean = jnp.mean(1.0 - lddt)

        iota_t_sub = jax.lax.broadcasted_iota(jnp.int32, (T, 1), 0)
        oh_cols = (iota_t_sub == tokr_ref[...]).astype(f32)
        u_atom = jnp.sum(oh_cols * ucol_ref[...], axis=0, keepdims=True)
        lig_atom = jnp.sum(oh_cols * ligcol_ref[...], axis=0, keepdims=True)

        cm = cmr_ref[...]
        t2 = t2_ref[...]
        w = (1.0 + u_atom) * (1.0 + lig_atom) * cm

        diff = [xl3_ref[c] - jnp.nan_to_num(xgt3_ref[c]) for c in range(3)]
        sq = diff[0] ** 2 + diff[1] ** 2 + diff[2] ** 2
        cm_sum = jnp.sum(cm)
        l_mse = (w * sq) / (3.0 * cm_sum + 1e-4)

        any_u = jnp.sum(u_atom) > 0.0
        te = t2 * (1.0 - 0.5 * u_atom)
        lgu = jnp.sum(_lam(te) * l_mse, axis=1, keepdims=True)
        rr = _lam(0.5 * t2) / _lam(t2)
        cmu = jnp.sum(cm * u_atom)
        cmn = jnp.sum(cm * (1.0 - u_atom))
        lgu = lgu * (cm_sum / (rr * cmu + cmn))
        lgp = _lam(t2) * jnp.sum(l_mse, axis=1, keepdims=True)
        lg = jnp.where(any_u, lgu, lgp)

        lp = w * (jnp.abs(diff[0]) + jnp.abs(diff[1]) + jnp.abs(diff[2]))
        usum = jnp.sum(u_atom)
        lpu = (jnp.sum(lp * u_atom, axis=1, keepdims=True)
               / (6.0 * usum + 1e-4)) * _lam(0.5 * t2)
        ltot = jnp.where(any_u, lg + 0.1 * lpu, lg)
        l_mse_total = jnp.mean(jnp.minimum(ltot, 2.0))

        out_ref[...] = jnp.broadcast_to(l_mse_total + lddt_mean, (1, 1))


def kernel(X_L, X_gt_L_in_input_frame, crd_mask_L, t, atom_to_token_map,
           is_original_unindexed_token, is_polar, is_ligand, is_virtual,
           is_sidechain, is_dna, is_rna):
    f32 = jnp.float32
    X_L = X_L.astype(f32)
    X_gt = X_gt_L_in_input_frame.astype(f32)
    xl3 = jnp.transpose(X_L, (2, 0, 1))
    xlr = jnp.transpose(X_L, (1, 0, 2)).reshape(L, 3 * D)
    xgt3 = jnp.transpose(X_gt, (2, 0, 1))
    xg0 = X_gt[0]
    tok = atom_to_token_map.astype(jnp.int32)
    tokc = tok.reshape(L, 1)
    tokr = tok.reshape(1, L)
    cm = crd_mask_L.astype(f32)
    cmc = cm.reshape(L, 1)
    cmr = cm.reshape(1, L)
    dna = is_dna.astype(f32).reshape(1, T)
    rna = is_rna.astype(f32).reshape(1, T)
    ucol = is_original_unindexed_token.astype(f32).reshape(T, 1)
    ligcol = is_ligand.astype(f32).reshape(T, 1)
    t2 = t.astype(f32).reshape(D, 1)

    rb_tab = jnp.array([rc[0] for rc in _TILES], dtype=jnp.int32)
    cb_tab = jnp.array([rc[1] for rc in _TILES], dtype=jnp.int32)

    full = lambda s: pl.BlockSpec(s, lambda i, rbt, cbt: (0,) * len(s))
    grid_spec = pltpu.PrefetchScalarGridSpec(
        num_scalar_prefetch=2,
        grid=(NT,),
        in_specs=[
            full((3, D, L)), full((L, 3 * D)), full((3, D, L)),
            full((L, 3)),
            full((L, 1)), full((1, L)), full((L, 1)), full((1, L)),
            full((1, T)), full((1, T)), full((T, 1)), full((T, 1)),
            full((D, 1)),
        ],
        out_specs=full((1, 1)),
        scratch_shapes=[
            pltpu.VMEM((D, L), f32),
            pltpu.VMEM((1, L), f32),
        ],
    )
    out = pl.pallas_call(
        _body,
        grid_spec=grid_spec,
        out_shape=jax.ShapeDtypeStruct((1, 1), f32),
    )(rb_tab, cb_tab, xl3, xlr, xgt3, xg0, tokc, tokr, cmc, cmr, dna,
      rna, ucol, ligcol, t2)
    return out[0, 0]


# speedup vs baseline: 1.3116x; 1.0839x over previous
import math

import jax
import jax.numpy as jnp
from jax.experimental import pallas as pl
from jax.experimental.pallas import tpu as pltpu

D = 16
L = 768
T = 96
BR = 128
BC = 128
NB = L // BR
NC = L // BC

_K = [math.exp(-0.5), math.exp(-1.0), math.exp(-2.0), math.exp(-4.0)]
_E1 = _K[0] + _K[1] + _K[2] + _K[3]
_E2 = sum(_K[i] * _K[j] for i in range(4) for j in range(i + 1, 4))
_E3 = sum(_K[i] * _K[j] * _K[l]
          for i in range(4) for j in range(i + 1, 4) for l in range(j + 1, 4))
_E4 = _K[0] * _K[1] * _K[2] * _K[3]
_P0, _P1, _P2, _P3 = 4.0, 3.0 * _E1, 2.0 * _E2, _E3


def _lam(s):
    return (s * s + 256.0) / ((s * 16.0) * (s * 16.0))


_TILES = [(rb, cb) for rb in range(NB) for cb in range(NC)
          if (cb + 1) * BC > rb * BR + 1]
NT = len(_TILES)


def _body(rbt_ref, cbt_ref, xl3_ref, xlr_ref, xgt3_ref, xgr_ref,
          auxc_ref, auxr_ref, auxtr_ref, auxtc_ref, t2_ref,
          out_ref, acc_s, acc_pm):
    i = pl.program_id(0)
    f32 = jnp.float32

    @pl.when(i == 0)
    def _init():
        acc_s[...] = jnp.zeros_like(acc_s)
        acc_pm[...] = jnp.zeros_like(acc_pm)

    if True:
        r0 = rbt_ref[i] * BR
        c0 = cbt_ref[i] * BC
        rows = jax.lax.broadcasted_iota(jnp.int32, (BR, 1), 0) + r0
        cols = jax.lax.broadcasted_iota(jnp.int32, (1, BC), 1) + c0

        gr = [jnp.nan_to_num(xgr_ref[pl.ds(r0, BR), c:c + 1])
              for c in range(3)]
        gc = [jnp.nan_to_num(xgt3_ref[c, 0:1, pl.ds(c0, BC)])
              for c in range(3)]
        gd = jnp.sqrt((gr[0] - gc[0]) ** 2 + (gr[1] - gc[1]) ** 2
                      + (gr[2] - gc[2]) ** 2)

        tokr_blk = auxc_ref[pl.ds(r0, BR), 1:2]
        iota_t_lane = jax.lax.broadcasted_iota(jnp.int32, (1, T), 1
                                               ).astype(f32)
        oh_rows = (tokr_blk == iota_t_lane).astype(f32)
        na_tok = jnp.minimum(auxtr_ref[0:1, :] + auxtr_ref[1:2, :], 1.0)
        na_rows = jnp.sum(oh_rows * na_tok, axis=1, keepdims=True)
        cutoff = jnp.where(na_rows > 0.0, 30.0, 15.0)

        cm_rows = auxc_ref[pl.ds(r0, BR), 0:1]
        pm = jnp.logical_and(gd > 0.0, gd < cutoff).astype(f32)
        pm = pm * cm_rows * auxr_ref[0:1, pl.ds(c0, BC)]
        pm = pm * (tokr_blk != auxr_ref[1:2, pl.ds(c0, BC)]).astype(f32)
        pm = pm * (cols > rows).astype(f32)
        acc_pm[:, pl.ds(c0, BC)] = (acc_pm[:, pl.ds(c0, BC)]
                                    + jnp.sum(pm, axis=0, keepdims=True))

        for d in range(D):
            pr = [xlr_ref[pl.ds(r0, BR), 3 * d + c:3 * d + c + 1]
                  for c in range(3)]
            co = [xl3_ref[c, d:d + 1, pl.ds(c0, BC)] for c in range(3)]
            dist2 = ((pr[0] - co[0]) ** 2 + (pr[1] - co[1]) ** 2
                     + (pr[2] - co[2]) ** 2)
            pred = jnp.sqrt(dist2)
            delta = jnp.abs(pred - gd + 1e-6)
            e = jnp.exp(delta)
            s = (1.0 / (1.0 + e * _K[0]) + 1.0 / (1.0 + e * _K[1])
                 + 1.0 / (1.0 + e * _K[2]) + 1.0 / (1.0 + e * _K[3]))
            acc_s[d:d + 1, pl.ds(c0, BC)] = (
                acc_s[d:d + 1, pl.ds(c0, BC)]
                + jnp.sum(s * pm, axis=0, keepdims=True))

    @pl.when(i == NT - 1)
    def _final():
        s_tot = jnp.sum(acc_s[...], axis=1, keepdims=True)
        pm_tot = jnp.sum(acc_pm[...])
        lddt = 0.25 * s_tot / (pm_tot + 1e-6)
        lddt_mean = jnp.mean(1.0 - lddt)

        iota_t_sub = jax.lax.broadcasted_iota(jnp.int32, (T, 1), 0
                                              ).astype(f32)
        oh_cols = (iota_t_sub == auxr_ref[1:2, :]).astype(f32)
        u_atom = jnp.sum(oh_cols * auxtc_ref[:, 0:1], axis=0, keepdims=True)
        lig_atom = jnp.sum(oh_cols * auxtc_ref[:, 1:2], axis=0,
                           keepdims=True)

        cm = auxr_ref[0:1, :]
        t2 = t2_ref[...]
        w = (1.0 + u_atom) * (1.0 + lig_atom) * cm

        diff = [xl3_ref[c] - jnp.nan_to_num(xgt3_ref[c]) for c in range(3)]
        sq = diff[0] ** 2 + diff[1] ** 2 + diff[2] ** 2
        cm_sum = jnp.sum(cm)
        l_mse = (w * sq) / (3.0 * cm_sum + 1e-4)

        any_u = jnp.sum(u_atom) > 0.0
        te = t2 * (1.0 - 0.5 * u_atom)
        lgu = jnp.sum(_lam(te) * l_mse, axis=1, keepdims=True)
        rr = _lam(0.5 * t2) / _lam(t2)
        cmu = jnp.sum(cm * u_atom)
        cmn = jnp.sum(cm * (1.0 - u_atom))
        lgu = lgu * (cm_sum / (rr * cmu + cmn))
        lgp = _lam(t2) * jnp.sum(l_mse, axis=1, keepdims=True)
        lg = jnp.where(any_u, lgu, lgp)

        lp = w * (jnp.abs(diff[0]) + jnp.abs(diff[1]) + jnp.abs(diff[2]))
        usum = jnp.sum(u_atom)
        lpu = (jnp.sum(lp * u_atom, axis=1, keepdims=True)
               / (6.0 * usum + 1e-4)) * _lam(0.5 * t2)
        ltot = jnp.where(any_u, lg + 0.1 * lpu, lg)
        l_mse_total = jnp.mean(jnp.minimum(ltot, 2.0))

        out_ref[...] = jnp.broadcast_to(l_mse_total + lddt_mean, (1, 1))


def kernel(X_L, X_gt_L_in_input_frame, crd_mask_L, t, atom_to_token_map,
           is_original_unindexed_token, is_polar, is_ligand, is_virtual,
           is_sidechain, is_dna, is_rna):
    f32 = jnp.float32
    X_L = X_L.astype(f32)
    X_gt = X_gt_L_in_input_frame.astype(f32)
    xl3 = jnp.transpose(X_L, (2, 0, 1))
    xlr = jnp.transpose(X_L, (1, 0, 2)).reshape(L, 3 * D)
    xgt3 = jnp.transpose(X_gt, (2, 0, 1))
    xg0 = X_gt[0]
    tokf = atom_to_token_map.astype(f32)
    cm = crd_mask_L.astype(f32)
    auxc = jnp.stack([cm, tokf], axis=1)
    auxr = jnp.stack([cm, tokf], axis=0)
    auxtr = jnp.stack([is_dna, is_rna], axis=0).astype(f32)
    auxtc = jnp.stack([is_original_unindexed_token, is_ligand],
                      axis=1).astype(f32)
    t2 = t.astype(f32).reshape(D, 1)

    rb_tab = jnp.array([rc[0] for rc in _TILES], dtype=jnp.int32)
    cb_tab = jnp.array([rc[1] for rc in _TILES], dtype=jnp.int32)

    full = lambda s: pl.BlockSpec(s, lambda i, rbt, cbt: (0,) * len(s))
    grid_spec = pltpu.PrefetchScalarGridSpec(
        num_scalar_prefetch=2,
        grid=(NT,),
        in_specs=[
            full((3, D, L)), full((L, 3 * D)), full((3, D, L)),
            full((L, 3)),
            full((L, 2)), full((2, L)), full((2, T)), full((T, 2)),
            full((D, 1)),
        ],
        out_specs=full((1, 1)),
        scratch_shapes=[
            pltpu.VMEM((D, L), f32),
            pltpu.VMEM((1, L), f32),
        ],
    )
    out = pl.pallas_call(
        _body,
        grid_spec=grid_spec,
        out_shape=jax.ShapeDtypeStruct((1, 1), f32),
    )(rb_tab, cb_tab, xl3, xlr, xgt3, xg0, auxc, auxr, auxtr, auxtc, t2)
    return out[0, 0]
